# baseline (speedup 1.0000x reference)
import functools

import jax
import jax.numpy as jnp
import numpy as np
from jax import lax
from jax.experimental import pallas as pl
from jax.experimental.pallas import tpu as pltpu
from jax.experimental.pallas import tpu_sc as plsc

N_NODES = 50000
FEAT = 128
EMBED = 128
S = 10

NC = 2
NS = 16
NW = NC * NS

B_PAD = 50176
BPW = B_PAD // NW
NCK = 32
CHUNKS = BPW // NCK
ROWS = (S + 1) * NCK


SLICES = ((0, 128), (128, 128), (256, ROWS - 256))


def _sc_body(aidx_hbm, tab_hbm, feat_hbm, self_out, sum_out,
             aidx0, aidx1, tn0, tn1, gath0, gath1, acc0, acc1,
             semA, semI, semW0, semW1):
    wid = lax.axis_index("s") * NC + lax.axis_index("c")
    base = wid * BPW
    abase = wid * CHUNKS * ROWS
    aidx_v = (aidx0, aidx1)
    tn_v = (tn0, tn1)
    gath_v = (gath0, gath1)
    acc_v = (acc0, acc1)
    semW = (semW0, semW1)


    def stage_addr(c, p):
        pltpu.sync_copy(aidx_hbm.at[pl.ds(abase + c * ROWS, ROWS)],
                        aidx_v[p])

    def ids_descs(p):
        return [pltpu.make_async_copy(tab_hbm.at[aidx_v[p]], tn_v[p], semI)]

    def feat_descs(p):
        return [pltpu.make_async_copy(feat_hbm.at[tn_v[p]], gath_v[p], semA)]

    def write_descs(c, p):
        dst = base + c * NCK
        return [pltpu.make_async_copy(gath_v[p].at[pl.ds(S * NCK, NCK)],
                                      self_out.at[pl.ds(dst, NCK)], semW[p]),
                pltpu.make_async_copy(acc_v[p],
                                      sum_out.at[pl.ds(dst, NCK)], semW[p])]

    stage_addr(0, 0)
    for d in ids_descs(0):
        d.start()
    for d in ids_descs(0):
        d.wait()
    for d in feat_descs(0):
        d.start()
    stage_addr(1, 1)
    for d in ids_descs(1):
        d.start()

    def do_iter(c, p):
        for d in feat_descs(p):
            d.wait()

        @pl.when(c + 1 < CHUNKS)
        def _():
            for d in ids_descs(1 - p):
                d.wait()

            @pl.when(c >= 1)
            def _():
                for d in write_descs(c - 1, 1 - p):
                    d.wait()

            for d in feat_descs(1 - p):
                d.start()

            @pl.when(c + 2 < CHUNKS)
            def _():
                stage_addr(c + 2, p)
                for d in ids_descs(p):
                    d.start()

        def red_row(r, _):
            for q in range(8):
                v = gath_v[p][r * S, pl.ds(q * 16, 16)]
                for j in range(1, S):
                    v = v + gath_v[p][r * S + j, pl.ds(q * 16, 16)]
                acc_v[p][r, pl.ds(q * 16, 16)] = v
            return 0

        lax.fori_loop(0, NCK, red_row, 0)

        for d in write_descs(c, p):
            d.start()

    def chunk(c, _):
        @pl.when(c % 2 == 0)
        def _():
            do_iter(c, 0)

        @pl.when(c % 2 == 1)
        def _():
            do_iter(c, 1)

        return 0

    lax.fori_loop(0, CHUNKS, chunk, 0)

    for d in write_descs(CHUNKS - 2, (CHUNKS - 2) % 2):
        d.wait()
    for d in write_descs(CHUNKS - 1, (CHUNKS - 1) % 2):
        d.wait()


@functools.partial(
    pl.kernel,
    out_type=(jax.ShapeDtypeStruct((B_PAD, FEAT), jnp.float32),
              jax.ShapeDtypeStruct((B_PAD, FEAT), jnp.float32)),
    mesh=plsc.VectorSubcoreMesh(core_axis_name="c", subcore_axis_name="s"),
    scratch_types=[
        pltpu.VMEM((ROWS,), jnp.int32),
        pltpu.VMEM((ROWS,), jnp.int32),
        pltpu.VMEM((ROWS,), jnp.int32),
        pltpu.VMEM((ROWS,), jnp.int32),
        pltpu.VMEM((ROWS, FEAT), jnp.float32),
        pltpu.VMEM((ROWS, FEAT), jnp.float32),
        pltpu.VMEM((NCK, FEAT), jnp.float32),
        pltpu.VMEM((NCK, FEAT), jnp.float32),
        pltpu.SemaphoreType.DMA,
        pltpu.SemaphoreType.DMA,
        pltpu.SemaphoreType.DMA,
        pltpu.SemaphoreType.DMA,
    ],
)
def _sc_gather(*refs):
    _sc_body(*refs)


def _tc_body(s_ref, n_ref, w1_ref, w2_ref, out_ref):
    dn = (((1,), (1,)), ((), ()))
    acc = lax.dot_general(s_ref[...], w1_ref[...], dn,
                          preferred_element_type=jnp.float32)
    acc += lax.dot_general(n_ref[...], w2_ref[...], dn,
                           preferred_element_type=jnp.float32)
    out_ref[...] = jnp.maximum(acc, 0.0)


def _tc_matmul(self_f, sum_f, w1, w2s):
    bt = 1024
    grid = B_PAD // bt
    return pl.pallas_call(
        _tc_body,
        grid=(grid,),
        in_specs=[
            pl.BlockSpec((bt, FEAT), lambda i: (i, 0)),
            pl.BlockSpec((bt, FEAT), lambda i: (i, 0)),
            pl.BlockSpec((EMBED, FEAT), lambda i: (0, 0)),
            pl.BlockSpec((EMBED, FEAT), lambda i: (0, 0)),
        ],
        out_specs=pl.BlockSpec((bt, EMBED), lambda i: (i, 0)),
        out_shape=jax.ShapeDtypeStruct((N_NODES, EMBED), jnp.float32),
    )(self_f, sum_f, w1, w2s)


_POS = np.concatenate([np.repeat(np.arange(NCK), S), np.arange(NCK)])
_OFF = np.concatenate([np.tile(np.arange(S) * N_NODES, NCK),
                       np.full(NCK, S * N_NODES)])


def kernel(nodes, features, neigh_idx, weight):
    nodes_p = jnp.concatenate(
        [nodes.astype(jnp.int32),
         jnp.zeros((B_PAD - N_NODES,), jnp.int32)])
    nodes_c = nodes_p.reshape(NW * CHUNKS, NCK)
    aidx = (jnp.take(nodes_c, jnp.asarray(_POS, jnp.int32), axis=1)
            + jnp.asarray(_OFF, jnp.int32)).reshape(-1)
    tab = jnp.concatenate([neigh_idx.astype(jnp.int32).T.reshape(-1),
                           jnp.arange(N_NODES, dtype=jnp.int32)])
    self_f, sum_f = _sc_gather(aidx, tab, features)
    w1 = weight[:, :FEAT]
    w2s = weight[:, FEAT:] * (1.0 / S)
    return _tc_matmul(self_f, sum_f, w1, w2s).T


# speedup vs baseline: 1.4087x; 1.4087x over previous
import functools

import jax
import jax.numpy as jnp
import numpy as np
from jax import lax
from jax.experimental import pallas as pl
from jax.experimental.pallas import tpu as pltpu
from jax.experimental.pallas import tpu_sc as plsc

N_NODES = 50000
FEAT = 128
EMBED = 128
S = 10

NC = 2
NS = 16
NW = NC * NS

B_PAD = 50176
BPW = B_PAD // NW
NCK = 32
CHUNKS = BPW // NCK
ROWS = (S + 1) * NCK


SLICES = ((0, 128), (128, 128), (256, ROWS - 256))


def _sc_body(nodes_hbm, tab_hbm, feat_hbm, self_out, sum_out,
             nodes_v, tn0, tn1, gath0, gath1, acc0, acc1,
             semA, semI, semW0, semW1):
    wid = lax.axis_index("s") * NC + lax.axis_index("c")
    base = wid * BPW
    tn_v = (tn0, tn1)
    gath_v = (gath0, gath1)
    acc_v = (acc0, acc1)
    semW = (semW0, semW1)


    def ids_descs(c, p):
        cb = pl.multiple_of(c * NCK, 8)
        return [pltpu.make_async_copy(
                    tab_hbm.at[pl.ds(j * N_NODES, N_NODES)]
                           .at[nodes_v.at[pl.ds(cb, NCK)]],
                    tn_v[p].at[pl.ds(j * NCK, NCK)], semI)
                for j in range(S)]

    def self_ids(c, p):
        cb = pl.multiple_of(c * NCK, 8)
        for h in range(NCK // 16):
            tn_v[p][pl.ds(S * NCK + h * 16, 16)] = (
                nodes_v[pl.ds(cb + h * 16, 16)])

    def feat_descs(p):
        return [pltpu.make_async_copy(feat_hbm.at[tn_v[p]], gath_v[p], semA)]

    def write_descs(c, p):
        dst = base + c * NCK
        return [pltpu.make_async_copy(gath_v[p].at[pl.ds(S * NCK, NCK)],
                                      self_out.at[pl.ds(dst, NCK)], semW[p]),
                pltpu.make_async_copy(acc_v[p],
                                      sum_out.at[pl.ds(dst, NCK)], semW[p])]

    pltpu.sync_copy(nodes_hbm.at[pl.ds(base, BPW)], nodes_v)
    for d in ids_descs(0, 0):
        d.start()
    self_ids(0, 0)
    for d in ids_descs(0, 0):
        d.wait()
    for d in feat_descs(0):
        d.start()
    for d in ids_descs(1, 1):
        d.start()
    self_ids(1, 1)

    def do_iter(c, p):
        for d in feat_descs(p):
            d.wait()

        @pl.when(c + 1 < CHUNKS)
        def _():
            for d in ids_descs(c + 1, 1 - p):
                d.wait()

            @pl.when(c >= 1)
            def _():
                for d in write_descs(c - 1, 1 - p):
                    d.wait()

            for d in feat_descs(1 - p):
                d.start()

            @pl.when(c + 2 < CHUNKS)
            def _():
                for d in ids_descs(c + 2, p):
                    d.start()
                self_ids(c + 2, p)

        def red_row(r, _):
            for q in range(8):
                v = gath_v[p][r, pl.ds(q * 16, 16)]
                for j in range(1, S):
                    v = v + gath_v[p][j * NCK + r, pl.ds(q * 16, 16)]
                acc_v[p][r, pl.ds(q * 16, 16)] = v
            return 0

        lax.fori_loop(0, NCK, red_row, 0)

        for d in write_descs(c, p):
            d.start()

    def chunk(c, _):
        @pl.when(c % 2 == 0)
        def _():
            do_iter(c, 0)

        @pl.when(c % 2 == 1)
        def _():
            do_iter(c, 1)

        return 0

    lax.fori_loop(0, CHUNKS, chunk, 0)

    for d in write_descs(CHUNKS - 2, (CHUNKS - 2) % 2):
        d.wait()
    for d in write_descs(CHUNKS - 1, (CHUNKS - 1) % 2):
        d.wait()


@functools.partial(
    pl.kernel,
    out_type=(jax.ShapeDtypeStruct((B_PAD, FEAT), jnp.float32),
              jax.ShapeDtypeStruct((B_PAD, FEAT), jnp.float32)),
    mesh=plsc.VectorSubcoreMesh(core_axis_name="c", subcore_axis_name="s"),
    scratch_types=[
        pltpu.VMEM((BPW,), jnp.int32),
        pltpu.VMEM((ROWS,), jnp.int32),
        pltpu.VMEM((ROWS,), jnp.int32),
        pltpu.VMEM((ROWS, FEAT), jnp.float32),
        pltpu.VMEM((ROWS, FEAT), jnp.float32),
        pltpu.VMEM((NCK, FEAT), jnp.float32),
        pltpu.VMEM((NCK, FEAT), jnp.float32),
        pltpu.SemaphoreType.DMA,
        pltpu.SemaphoreType.DMA,
        pltpu.SemaphoreType.DMA,
        pltpu.SemaphoreType.DMA,
    ],
)
def _sc_gather(*refs):
    _sc_body(*refs)


def _tc_body(s_ref, n_ref, w1_ref, w2_ref, out_ref):
    dn = (((1,), (1,)), ((), ()))
    acc = lax.dot_general(s_ref[...], w1_ref[...], dn,
                          preferred_element_type=jnp.float32)
    acc += lax.dot_general(n_ref[...], w2_ref[...], dn,
                           preferred_element_type=jnp.float32)
    out_ref[...] = jnp.maximum(acc, 0.0)


def _tc_matmul(self_f, sum_f, w1, w2s):
    bt = 1024
    grid = B_PAD // bt
    return pl.pallas_call(
        _tc_body,
        grid=(grid,),
        in_specs=[
            pl.BlockSpec((bt, FEAT), lambda i: (i, 0)),
            pl.BlockSpec((bt, FEAT), lambda i: (i, 0)),
            pl.BlockSpec((EMBED, FEAT), lambda i: (0, 0)),
            pl.BlockSpec((EMBED, FEAT), lambda i: (0, 0)),
        ],
        out_specs=pl.BlockSpec((bt, EMBED), lambda i: (i, 0)),
        out_shape=jax.ShapeDtypeStruct((N_NODES, EMBED), jnp.float32),
    )(self_f, sum_f, w1, w2s)


def kernel(nodes, features, neigh_idx, weight):
    nodes_p = jnp.concatenate(
        [nodes.astype(jnp.int32),
         jnp.zeros((B_PAD - N_NODES,), jnp.int32)])
    tab = neigh_idx.astype(jnp.int32).T.reshape(-1)
    self_f, sum_f = _sc_gather(nodes_p, tab, features)
    w1 = weight[:, :FEAT]
    w2s = weight[:, FEAT:] * (1.0 / S)
    return _tc_matmul(self_f, sum_f, w1, w2s).T
